# baseline (device time: 29860 ns/iter reference)
import jax
import jax.numpy as jnp
from jax import lax
from jax.experimental import pallas as pl
from jax.experimental.pallas import tpu as pltpu

N_DEV = 4
N_LAYERS = 3


def kernel(x, Win0, Wout0, Win1, Wout1, Win2, Wout2):
    b, d = x.shape

    def body(
        x_ref,
        win0_ref,
        wout0_ref,
        win1_ref,
        wout1_ref,
        win2_ref,
        wout2_ref,
        out_ref,
        send_buf,
        comm_ref,
        send_sems,
        recv_sems,
    ):
        my = lax.axis_index("i")
        wins = [win0_ref, win1_ref, win2_ref]
        wouts = [wout0_ref, wout1_ref, wout2_ref]

        x_val = x_ref[:, :].astype(jnp.bfloat16)
        for l in range(N_LAYERS):
            h = jnp.dot(
                x_val, wins[l][:, :].astype(jnp.bfloat16),
                preferred_element_type=jnp.float32,
            )
            h = jnp.maximum(h, 0.0).astype(jnp.bfloat16)
            partial = jnp.dot(
                h, wouts[l][:, :].astype(jnp.bfloat16),
                preferred_element_type=jnp.float32,
            )
            send_buf[l, :, :] = partial.astype(jnp.bfloat16)

            rdmas = []
            for off in range(1, N_DEV):
                peer = lax.rem(my + off, N_DEV)
                rdma = pltpu.make_async_remote_copy(
                    src_ref=send_buf.at[l],
                    dst_ref=comm_ref.at[l, N_DEV - 1 - off],
                    send_sem=send_sems.at[l, off - 1],
                    recv_sem=recv_sems.at[l, N_DEV - 1 - off],
                    device_id=(peer,),
                    device_id_type=pl.DeviceIdType.MESH,
                )
                rdma.start()
                rdmas.append(rdma)

            total = partial
            for off in range(1, N_DEV):
                rdmas[off - 1].wait_recv()
                total = total + comm_ref[l, N_DEV - 1 - off, :, :].astype(
                    jnp.float32
                )
            for r in rdmas:
                r.wait_send()

            if l == N_LAYERS - 1:
                out_ref[:, :] = total
            else:
                x_val = total.astype(jnp.bfloat16)

    return pl.pallas_call(
        body,
        out_shape=jax.ShapeDtypeStruct((b, d), jnp.float32),
        in_specs=[pl.BlockSpec(memory_space=pltpu.VMEM)] * 7,
        out_specs=pl.BlockSpec(memory_space=pltpu.VMEM),
        scratch_shapes=[
            pltpu.VMEM((N_LAYERS, b, d), jnp.bfloat16),
            pltpu.VMEM((N_LAYERS, N_DEV - 1, b, d), jnp.bfloat16),
            pltpu.SemaphoreType.DMA((N_LAYERS, N_DEV - 1)),
            pltpu.SemaphoreType.DMA((N_LAYERS, N_DEV - 1)),
        ],
    )(x, Win0, Wout0, Win1, Wout1, Win2, Wout2)


# device time: 26578 ns/iter; 1.1235x vs baseline; 1.1235x over previous
import jax
import jax.numpy as jnp
from jax import lax
from jax.experimental import pallas as pl
from jax.experimental.pallas import tpu as pltpu

N_DEV = 4
N_LAYERS = 3


def kernel(x, Win0, Wout0, Win1, Wout1, Win2, Wout2):
    b, d = x.shape

    def body(
        x_ref,
        win0_ref,
        wout0_ref,
        win1_ref,
        wout1_ref,
        win2_ref,
        wout2_ref,
        out_ref,
        send_buf,
        comm_ref,
        send_sems,
        recv_sems,
    ):
        my = lax.axis_index("i")
        wins = [win0_ref, win1_ref, win2_ref]
        wouts = [wout0_ref, wout1_ref, wout2_ref]

        barrier_sem = pltpu.get_barrier_semaphore()
        for off in range(1, N_DEV):
            pl.semaphore_signal(
                barrier_sem, inc=1,
                device_id=(lax.rem(my + off, N_DEV),),
                device_id_type=pl.DeviceIdType.MESH,
            )

        x_val = x_ref[:, :].astype(jnp.bfloat16)
        all_rdmas = []
        for l in range(N_LAYERS):
            h = jnp.dot(
                x_val, wins[l][:, :].astype(jnp.bfloat16),
                preferred_element_type=jnp.float32,
            )
            h = jnp.maximum(h, 0.0).astype(jnp.bfloat16)
            partial = jnp.dot(
                h, wouts[l][:, :].astype(jnp.bfloat16),
                preferred_element_type=jnp.float32,
            )
            send_buf[l, :, :] = partial.astype(jnp.bfloat16)

            if l == 0:
                pl.semaphore_wait(barrier_sem, N_DEV - 1)

            rdmas = []
            for off in range(1, N_DEV):
                peer = lax.rem(my + off, N_DEV)
                rdma = pltpu.make_async_remote_copy(
                    src_ref=send_buf.at[l],
                    dst_ref=comm_ref.at[l, N_DEV - 1 - off],
                    send_sem=send_sems.at[l, off - 1],
                    recv_sem=recv_sems.at[l, N_DEV - 1 - off],
                    device_id=(peer,),
                    device_id_type=pl.DeviceIdType.MESH,
                )
                rdma.start()
                rdmas.append(rdma)

            all_rdmas.extend(rdmas)

            total = partial
            for off in range(1, N_DEV):
                rdmas[off - 1].wait_recv()
                total = total + comm_ref[l, N_DEV - 1 - off, :, :].astype(
                    jnp.float32
                )

            if l == N_LAYERS - 1:
                out_ref[:, :] = total
            else:
                x_val = total.astype(jnp.bfloat16)

        for r in all_rdmas:
            r.wait_send()

    return pl.pallas_call(
        body,
        out_shape=jax.ShapeDtypeStruct((b, d), jnp.float32),
        in_specs=[pl.BlockSpec(memory_space=pltpu.VMEM)] * 7,
        out_specs=pl.BlockSpec(memory_space=pltpu.VMEM),
        scratch_shapes=[
            pltpu.VMEM((N_LAYERS, b, d), jnp.bfloat16),
            pltpu.VMEM((N_LAYERS, N_DEV - 1, b, d), jnp.bfloat16),
            pltpu.SemaphoreType.DMA((N_LAYERS, N_DEV - 1)),
            pltpu.SemaphoreType.DMA((N_LAYERS, N_DEV - 1)),
        ],
        compiler_params=pltpu.CompilerParams(collective_id=0),
    )(x, Win0, Wout0, Win1, Wout1, Win2, Wout2)


# device time: 8239 ns/iter; 3.6242x vs baseline; 3.2259x over previous
import jax
import jax.numpy as jnp
from jax import lax
from jax.experimental import pallas as pl
from jax.experimental.pallas import tpu as pltpu

N_DEV = 4
N_LAYERS = 3
SEND_ORDER = [2, 1, 3]


def kernel(x, Win0, Wout0, Win1, Wout1, Win2, Wout2):
    b, d = x.shape
    _, hdim = Win0.shape

    def body(
        x_ref,
        win0_ref,
        wout0_ref,
        win1_ref,
        wout1_ref,
        win2_ref,
        wout2_ref,
        out_ref,
        send_buf,
        comm_ref,
        win_bf,
        wout_bf,
        send_sems,
        recv_sems,
    ):
        my = lax.axis_index("i")
        wins = [win0_ref, win1_ref, win2_ref]
        wouts = [wout0_ref, wout1_ref, wout2_ref]

        barrier_sem = pltpu.get_barrier_semaphore()
        for off in range(1, N_DEV):
            pl.semaphore_signal(
                barrier_sem, inc=1,
                device_id=(lax.rem(my + off, N_DEV),),
                device_id_type=pl.DeviceIdType.MESH,
            )

        win_bf[0, :, :] = wins[0][:, :].astype(jnp.bfloat16)
        wout_bf[0, :, :] = wouts[0][:, :].astype(jnp.bfloat16)

        x_val = x_ref[:, :].astype(jnp.bfloat16)
        all_rdmas = []
        for l in range(N_LAYERS):
            h = jnp.dot(
                x_val, win_bf[l, :, :], preferred_element_type=jnp.float32
            )
            h = jnp.maximum(h, 0.0).astype(jnp.bfloat16)
            partial = jnp.dot(
                h, wout_bf[l, :, :], preferred_element_type=jnp.float32
            )
            send_buf[l, :, :] = partial.astype(jnp.bfloat16)

            if l == 0:
                pl.semaphore_wait(barrier_sem, N_DEV - 1)

            rdmas = []
            for off in SEND_ORDER:
                peer = lax.rem(my + off, N_DEV)
                rdma = pltpu.make_async_remote_copy(
                    src_ref=send_buf.at[l],
                    dst_ref=comm_ref.at[l, N_DEV - 1 - off],
                    send_sem=send_sems.at[l, off - 1],
                    recv_sem=recv_sems.at[l, N_DEV - 1 - off],
                    device_id=(peer,),
                    device_id_type=pl.DeviceIdType.MESH,
                )
                rdma.start()
                rdmas.append(rdma)

            all_rdmas.extend(rdmas)

            if l + 1 < N_LAYERS:
                win_bf[l + 1, :, :] = wins[l + 1][:, :].astype(jnp.bfloat16)
                wout_bf[l + 1, :, :] = wouts[l + 1][:, :].astype(jnp.bfloat16)

            total = partial
            for i, off in enumerate(SEND_ORDER):
                rdmas[i].wait_recv()
                total = total + comm_ref[l, N_DEV - 1 - off, :, :].astype(
                    jnp.float32
                )

            if l == N_LAYERS - 1:
                out_ref[:, :] = total
            else:
                x_val = total.astype(jnp.bfloat16)

        for r in all_rdmas:
            r.wait_send()

    return pl.pallas_call(
        body,
        out_shape=jax.ShapeDtypeStruct((b, d), jnp.float32),
        in_specs=[pl.BlockSpec(memory_space=pltpu.VMEM)] * 7,
        out_specs=pl.BlockSpec(memory_space=pltpu.VMEM),
        scratch_shapes=[
            pltpu.VMEM((N_LAYERS, b, d), jnp.bfloat16),
            pltpu.VMEM((N_LAYERS, N_DEV - 1, b, d), jnp.bfloat16),
            pltpu.VMEM((N_LAYERS, d, hdim), jnp.bfloat16),
            pltpu.VMEM((N_LAYERS, hdim, d), jnp.bfloat16),
            pltpu.SemaphoreType.DMA((N_LAYERS, N_DEV - 1)),
            pltpu.SemaphoreType.DMA((N_LAYERS, N_DEV - 1)),
        ],
        compiler_params=pltpu.CompilerParams(collective_id=0),
    )(x, Win0, Wout0, Win1, Wout1, Win2, Wout2)
